# baseline (speedup 1.0000x reference)
import functools

import jax
import jax.numpy as jnp
from jax import lax
from jax.experimental import pallas as pl
from jax.experimental.pallas import tpu as pltpu
from jax.experimental.pallas import tpu_sc as plsc

_NALLELES = 10
_D = 32
_L = 16
_NC = 2
_NS = 16
_NW = _NC * _NS

_GATHER_DNUMS = lax.GatherDimensionNumbers(
    offset_dims=(), collapsed_slice_dims=(0,), start_index_map=(0,))


def _lane_gather(src, idx):
  return lax.gather(src, idx[:, None], _GATHER_DNUMS, slice_sizes=(1,),
                    mode=lax.GatherScatterMode.PROMISE_IN_BOUNDS)


def _sc_embed(pos_flat, al2_flat, table, n_rows, chunk):
  per_w = n_rows // _NW
  n_chunks = per_w // chunk
  assert per_w % chunk == 0 and n_chunks % 2 == 0 and n_chunks >= 8

  mesh = plsc.VectorSubcoreMesh(core_axis_name="c", subcore_axis_name="s")

  @functools.partial(
      pl.kernel,
      mesh=mesh,
      out_type=jax.ShapeDtypeStruct((n_rows, _D), jnp.float32),
      compiler_params=pltpu.CompilerParams(use_tc_tiling_on_sc=False),
      scratch_types=(
          [pltpu.VMEM((chunk,), jnp.int32)] * 2
          + [pltpu.VMEM((2 * chunk,), jnp.int32)] * 4
          + [pltpu.VMEM((2 * chunk, _D), jnp.float32)] * 2
          + [pltpu.VMEM((chunk, _D), jnp.float32)] * 2
          + [pltpu.SemaphoreType.DMA] * 6),
  )
  def k(pos_hbm, al_hbm, table_hbm, out_hbm,
        pos0, pos1, al0, al1, ix0, ix1, g0, g1, o0, o1,
        isem0, isem1, gsem0, gsem1, wsem0, wsem1):
    pos_v = (pos0, pos1)
    al_v = (al0, al1)
    ix_v = (ix0, ix1)
    g_v = (g0, g1)
    o_v = (o0, o1)
    isem = (isem0, isem1)
    gsem = (gsem0, gsem1)
    wsem = (wsem0, wsem1)

    wid = lax.axis_index("s") * _NC + lax.axis_index("c")
    w_base = wid * per_w

    lane = lax.iota(jnp.int32, _L)
    dup_lo = lax.shift_right_logical(lane, 1)
    dup_hi = dup_lo + 8

    def issue_in(t, b):
      base = w_base + t * chunk
      pltpu.async_copy(pos_hbm.at[pl.ds(base, chunk)], pos_v[b], isem[b])
      pltpu.async_copy(al_hbm.at[pl.ds(2 * base, 2 * chunk)], al_v[b],
                       isem[b])

    def wait_in(b):
      pltpu.make_async_copy(pos_hbm.at[pl.ds(0, chunk)], pos_v[b],
                            isem[b]).wait()
      pltpu.make_async_copy(al_hbm.at[pl.ds(0, 2 * chunk)], al_v[b],
                            isem[b]).wait()

    def compute_idx(b):
      def body(j, _):
        s = j * _L
        p = pos_v[b][pl.ds(s, _L)] * _NALLELES
        lo = _lane_gather(p, dup_lo)
        hi = _lane_gather(p, dup_hi)
        ix_v[b][pl.ds(2 * s, _L)] = lo + al_v[b][pl.ds(2 * s, _L)]
        ix_v[b][pl.ds(2 * s + _L, _L)] = hi + al_v[b][pl.ds(2 * s + _L, _L)]
        return 0

      lax.fori_loop(0, chunk // _L, body, 0, unroll=4)

    def issue_gather(b):
      pltpu.async_copy(table_hbm.at[ix_v[b]], g_v[b], gsem[b])

    def wait_gather(b):
      pltpu.make_async_copy(out_hbm.at[pl.ds(0, 2 * chunk)], g_v[b],
                            gsem[b]).wait()

    def add_rows(b):
      def body(i, _):
        i2 = 2 * i
        o_v[b][i, pl.ds(0, _L)] = (
            g_v[b][i2, pl.ds(0, _L)] + g_v[b][i2 + 1, pl.ds(0, _L)])
        o_v[b][i, pl.ds(_L, _L)] = (
            g_v[b][i2, pl.ds(_L, _L)] + g_v[b][i2 + 1, pl.ds(_L, _L)])
        return 0

      lax.fori_loop(0, chunk, body, 0, unroll=4)

    def issue_wb(t, b):
      base = w_base + t * chunk
      pltpu.async_copy(o_v[b], out_hbm.at[pl.ds(base, chunk)], wsem[b])

    def wait_wb(b):
      pltpu.make_async_copy(o_v[b], out_hbm.at[pl.ds(0, chunk)],
                            wsem[b]).wait()

    issue_in(0, 0)
    issue_in(1, 1)
    wait_in(0)
    compute_idx(0)
    issue_in(2, 0)
    issue_gather(0)
    wait_in(1)
    compute_idx(1)
    issue_in(3, 1)
    issue_gather(1)
    wait_gather(0)
    add_rows(0)
    issue_wb(0, 0)
    wait_in(0)
    compute_idx(0)
    issue_in(4, 0)
    issue_gather(0)
    wait_gather(1)
    add_rows(1)
    issue_wb(1, 1)
    wait_in(1)
    compute_idx(1)
    issue_in(5, 1)
    issue_gather(1)
    wait_gather(0)
    wait_wb(0)
    add_rows(0)
    issue_wb(2, 0)

    def pair_body(g, _):
      for b in range(2):
        t = 2 * g + b
        wait_in(b)
        compute_idx(b)
        issue_in(t + 2, b)
        issue_gather(b)
        wait_gather(1 - b)
        wait_wb(1 - b)
        add_rows(1 - b)
        issue_wb(t - 1, 1 - b)
      return 0

    lax.fori_loop(2, n_chunks // 2 - 1, pair_body, 0)

    tl = n_chunks - 2
    wait_in(0)
    compute_idx(0)
    issue_gather(0)
    wait_gather(1)
    wait_wb(1)
    add_rows(1)
    issue_wb(tl - 1, 1)

    wait_in(1)
    compute_idx(1)
    issue_gather(1)
    wait_gather(0)
    wait_wb(0)
    add_rows(0)
    issue_wb(tl, 0)

    wait_gather(1)
    wait_wb(1)
    add_rows(1)
    issue_wb(tl + 1, 1)
    wait_wb(0)
    wait_wb(1)

  return k(pos_flat, al2_flat, table)


def kernel(alleles, positions, table):
  b, s, _ = alleles.shape
  n = b * s
  pos_flat = positions.reshape(n)
  al2 = alleles.reshape(2 * n)
  out = _sc_embed(pos_flat, al2, table, n, 640)
  return out.reshape(b, s, _D)


# speedup vs baseline: 1.7981x; 1.7981x over previous
import functools

import jax
import jax.numpy as jnp
from jax import lax
from jax.experimental import pallas as pl
from jax.experimental.pallas import tpu as pltpu
from jax.experimental.pallas import tpu_sc as plsc

_NALLELES = 10
_D = 32
_L = 16
_NC = 2
_NS = 16
_NW = _NC * _NS

_GATHER_DNUMS = lax.GatherDimensionNumbers(
    offset_dims=(), collapsed_slice_dims=(0,), start_index_map=(0,))


def _lane_gather(src, idx):
  return lax.gather(src, idx[:, None], _GATHER_DNUMS, slice_sizes=(1,),
                    mode=lax.GatherScatterMode.PROMISE_IN_BOUNDS)


def _sc_embed(packed, table, n_rows, chunk):
  per_w = n_rows // _NW
  n_chunks = per_w // chunk
  assert per_w % chunk == 0 and n_chunks % 2 == 0 and n_chunks >= 8

  mesh = plsc.VectorSubcoreMesh(core_axis_name="c", subcore_axis_name="s")

  @functools.partial(
      pl.kernel,
      mesh=mesh,
      out_type=jax.ShapeDtypeStruct((n_rows, _D), jnp.float32),
      compiler_params=pltpu.CompilerParams(use_tc_tiling_on_sc=False),
      scratch_types=(
          [pltpu.VMEM((chunk,), jnp.int32)] * 2
          + [pltpu.VMEM((2 * chunk,), jnp.int32)] * 2
          + [pltpu.VMEM((2 * chunk, _D), jnp.float32)] * 2
          + [pltpu.VMEM((chunk, _D), jnp.float32)] * 2
          + [pltpu.SemaphoreType.DMA] * 6),
  )
  def k(w_hbm, table_hbm, out_hbm,
        w0, w1, ix0, ix1, g0, g1, o0, o1,
        isem0, isem1, gsem0, gsem1, wsem0, wsem1):
    w_v = (w0, w1)
    ix_v = (ix0, ix1)
    g_v = (g0, g1)
    o_v = (o0, o1)
    isem = (isem0, isem1)
    gsem = (gsem0, gsem1)
    wsem = (wsem0, wsem1)

    wid = lax.axis_index("s") * _NC + lax.axis_index("c")
    w_base = wid * per_w

    lane = lax.iota(jnp.int32, _L)
    dup_lo = lax.shift_right_logical(lane, 1)
    dup_hi = dup_lo + 8
    a_shift = (1 - (lane & 1)) * 4

    def issue_in(t, b):
      base = w_base + t * chunk
      pltpu.async_copy(w_hbm.at[pl.ds(base, chunk)], w_v[b], isem[b])

    def wait_in(b):
      pltpu.make_async_copy(w_hbm.at[pl.ds(0, chunk)], w_v[b],
                            isem[b]).wait()

    def compute_idx(b):
      def body(j, _):
        s = j * _L
        w = w_v[b][pl.ds(s, _L)]
        wlo = _lane_gather(w, dup_lo)
        whi = _lane_gather(w, dup_hi)
        ixlo = (lax.shift_right_logical(wlo, 8) * _NALLELES
                + (lax.shift_right_logical(wlo, a_shift) & 15))
        ixhi = (lax.shift_right_logical(whi, 8) * _NALLELES
                + (lax.shift_right_logical(whi, a_shift) & 15))
        ix_v[b][pl.ds(2 * s, _L)] = ixlo
        ix_v[b][pl.ds(2 * s + _L, _L)] = ixhi
        return 0

      lax.fori_loop(0, chunk // _L, body, 0, unroll=4)

    def issue_gather(b):
      pltpu.async_copy(table_hbm.at[ix_v[b]], g_v[b], gsem[b])

    def wait_gather(b):
      pltpu.make_async_copy(out_hbm.at[pl.ds(0, 2 * chunk)], g_v[b],
                            gsem[b]).wait()

    def add_rows(b):
      def body(i, _):
        i2 = 2 * i
        o_v[b][i, pl.ds(0, _L)] = (
            g_v[b][i2, pl.ds(0, _L)] + g_v[b][i2 + 1, pl.ds(0, _L)])
        o_v[b][i, pl.ds(_L, _L)] = (
            g_v[b][i2, pl.ds(_L, _L)] + g_v[b][i2 + 1, pl.ds(_L, _L)])
        return 0

      lax.fori_loop(0, chunk, body, 0, unroll=4)

    def issue_wb(t, b):
      base = w_base + t * chunk
      pltpu.async_copy(o_v[b], out_hbm.at[pl.ds(base, chunk)], wsem[b])

    def wait_wb(b):
      pltpu.make_async_copy(o_v[b], out_hbm.at[pl.ds(0, chunk)],
                            wsem[b]).wait()

    issue_in(0, 0)
    issue_in(1, 1)
    wait_in(0)
    compute_idx(0)
    issue_in(2, 0)
    issue_gather(0)
    wait_in(1)
    compute_idx(1)
    issue_in(3, 1)
    issue_gather(1)
    wait_gather(0)
    add_rows(0)
    issue_wb(0, 0)
    wait_in(0)
    compute_idx(0)
    issue_in(4, 0)
    issue_gather(0)
    wait_gather(1)
    add_rows(1)
    issue_wb(1, 1)
    wait_in(1)
    compute_idx(1)
    issue_in(5, 1)
    issue_gather(1)
    wait_gather(0)
    wait_wb(0)
    add_rows(0)
    issue_wb(2, 0)

    def pair_body(g, _):
      for b in range(2):
        t = 2 * g + b
        wait_in(b)
        compute_idx(b)
        issue_in(t + 2, b)
        issue_gather(b)
        wait_gather(1 - b)
        wait_wb(1 - b)
        add_rows(1 - b)
        issue_wb(t - 1, 1 - b)
      return 0

    lax.fori_loop(2, n_chunks // 2 - 1, pair_body, 0)

    tl = n_chunks - 2
    wait_in(0)
    compute_idx(0)
    issue_gather(0)
    wait_gather(1)
    wait_wb(1)
    add_rows(1)
    issue_wb(tl - 1, 1)

    wait_in(1)
    compute_idx(1)
    issue_gather(1)
    wait_gather(0)
    wait_wb(0)
    add_rows(0)
    issue_wb(tl, 0)

    wait_gather(1)
    wait_wb(1)
    add_rows(1)
    issue_wb(tl + 1, 1)
    wait_wb(0)
    wait_wb(1)

  return k(packed, table)


def kernel(alleles, positions, table):
  b, s, _ = alleles.shape
  n = b * s
  packed = (
      lax.shift_left(positions.astype(jnp.int32), 8)
      | lax.shift_left(alleles[:, :, 0].astype(jnp.int32), 4)
      | alleles[:, :, 1].astype(jnp.int32)
  ).reshape(n)
  out = _sc_embed(packed, table, n, 640)
  return out.reshape(b, s, _D)


# speedup vs baseline: 1.7987x; 1.0003x over previous
import functools

import jax
import jax.numpy as jnp
from jax import lax
from jax.experimental import pallas as pl
from jax.experimental.pallas import tpu as pltpu
from jax.experimental.pallas import tpu_sc as plsc

_NALLELES = 10
_D = 32
_L = 16
_NC = 2
_NS = 16
_NW = _NC * _NS

_GATHER_DNUMS = lax.GatherDimensionNumbers(
    offset_dims=(), collapsed_slice_dims=(0,), start_index_map=(0,))


def _lane_gather(src, idx):
  return lax.gather(src, idx[:, None], _GATHER_DNUMS, slice_sizes=(1,),
                    mode=lax.GatherScatterMode.PROMISE_IN_BOUNDS)


def _sc_embed(packed, table, n_rows, chunk):
  per_w = n_rows // _NW
  n_chunks = per_w // chunk
  assert per_w % chunk == 0 and n_chunks % 2 == 0 and n_chunks >= 8

  mesh = plsc.VectorSubcoreMesh(core_axis_name="c", subcore_axis_name="s")

  @functools.partial(
      pl.kernel,
      mesh=mesh,
      out_type=jax.ShapeDtypeStruct((n_rows * _D,), jnp.float32),
      compiler_params=pltpu.CompilerParams(use_tc_tiling_on_sc=False),
      scratch_types=(
          [pltpu.VMEM((chunk,), jnp.int32)] * 2
          + [pltpu.VMEM((2 * chunk,), jnp.int32)] * 2
          + [pltpu.VMEM((2 * chunk, _D), jnp.float32)] * 2
          + [pltpu.VMEM((chunk * _D,), jnp.float32)] * 2
          + [pltpu.SemaphoreType.DMA] * 6),
  )
  def k(w_hbm, table_hbm, out_hbm,
        w0, w1, ix0, ix1, g0, g1, o0, o1,
        isem0, isem1, gsem0, gsem1, wsem0, wsem1):
    w_v = (w0, w1)
    ix_v = (ix0, ix1)
    g_v = (g0, g1)
    o_v = (o0, o1)
    isem = (isem0, isem1)
    gsem = (gsem0, gsem1)
    wsem = (wsem0, wsem1)

    wid = lax.axis_index("s") * _NC + lax.axis_index("c")
    w_base = wid * per_w

    lane = lax.iota(jnp.int32, _L)
    dup_lo = lax.shift_right_logical(lane, 1)
    dup_hi = dup_lo + 8
    a_shift = (1 - (lane & 1)) * 4

    def issue_in(t, b):
      base = w_base + t * chunk
      pltpu.async_copy(w_hbm.at[pl.ds(base, chunk)], w_v[b], isem[b])

    def wait_in(b):
      pltpu.make_async_copy(w_hbm.at[pl.ds(0, chunk)], w_v[b],
                            isem[b]).wait()

    def compute_idx(b):
      def body(j, _):
        s = j * _L
        w = w_v[b][pl.ds(s, _L)]
        wlo = _lane_gather(w, dup_lo)
        whi = _lane_gather(w, dup_hi)
        ixlo = (lax.shift_right_logical(wlo, 8) * _NALLELES
                + (lax.shift_right_logical(wlo, a_shift) & 15))
        ixhi = (lax.shift_right_logical(whi, 8) * _NALLELES
                + (lax.shift_right_logical(whi, a_shift) & 15))
        ix_v[b][pl.ds(2 * s, _L)] = ixlo
        ix_v[b][pl.ds(2 * s + _L, _L)] = ixhi
        return 0

      lax.fori_loop(0, chunk // _L, body, 0, unroll=4)

    def issue_gather(b):
      pltpu.async_copy(table_hbm.at[ix_v[b]], g_v[b], gsem[b])

    def wait_gather(b):
      pltpu.make_async_copy(table_hbm.at[pl.ds(0, 2 * chunk)], g_v[b],
                            gsem[b]).wait()

    def add_rows(b):
      def body(i, _):
        i2 = 2 * i
        o_v[b][pl.ds(_D * i, _L)] = (
            g_v[b][i2, pl.ds(0, _L)] + g_v[b][i2 + 1, pl.ds(0, _L)])
        o_v[b][pl.ds(_D * i + _L, _L)] = (
            g_v[b][i2, pl.ds(_L, _L)] + g_v[b][i2 + 1, pl.ds(_L, _L)])
        return 0

      lax.fori_loop(0, chunk, body, 0, unroll=4)

    def issue_wb(t, b):
      base = w_base + t * chunk
      pltpu.async_copy(o_v[b], out_hbm.at[pl.ds(base * _D, chunk * _D)],
                       wsem[b])

    def wait_wb(b):
      pltpu.make_async_copy(o_v[b], out_hbm.at[pl.ds(0, chunk * _D)],
                            wsem[b]).wait()

    issue_in(0, 0)
    issue_in(1, 1)
    wait_in(0)
    compute_idx(0)
    issue_in(2, 0)
    issue_gather(0)
    wait_in(1)
    compute_idx(1)
    issue_in(3, 1)
    issue_gather(1)
    wait_gather(0)
    add_rows(0)
    issue_wb(0, 0)
    wait_in(0)
    compute_idx(0)
    issue_in(4, 0)
    issue_gather(0)
    wait_gather(1)
    add_rows(1)
    issue_wb(1, 1)
    wait_in(1)
    compute_idx(1)
    issue_in(5, 1)
    issue_gather(1)
    wait_gather(0)
    wait_wb(0)
    add_rows(0)
    issue_wb(2, 0)

    def pair_body(g, _):
      for b in range(2):
        t = 2 * g + b
        wait_in(b)
        compute_idx(b)
        issue_in(t + 2, b)
        issue_gather(b)
        wait_gather(1 - b)
        wait_wb(1 - b)
        add_rows(1 - b)
        issue_wb(t - 1, 1 - b)
      return 0

    lax.fori_loop(2, n_chunks // 2 - 1, pair_body, 0)

    tl = n_chunks - 2
    wait_in(0)
    compute_idx(0)
    issue_gather(0)
    wait_gather(1)
    wait_wb(1)
    add_rows(1)
    issue_wb(tl - 1, 1)

    wait_in(1)
    compute_idx(1)
    issue_gather(1)
    wait_gather(0)
    wait_wb(0)
    add_rows(0)
    issue_wb(tl, 0)

    wait_gather(1)
    wait_wb(1)
    add_rows(1)
    issue_wb(tl + 1, 1)
    wait_wb(0)
    wait_wb(1)

  return k(packed, table)


def kernel(alleles, positions, table):
  b, s, _ = alleles.shape
  n = b * s
  packed = (
      lax.shift_left(positions.astype(jnp.int32), 8)
      | lax.shift_left(alleles[:, :, 0].astype(jnp.int32), 4)
      | alleles[:, :, 1].astype(jnp.int32)
  ).reshape(n)
  out = _sc_embed(packed, table, n, 640)
  return out.reshape(b, s, _D)


# speedup vs baseline: 1.7993x; 1.0004x over previous
import functools

import jax
import jax.numpy as jnp
from jax import lax
from jax.experimental import pallas as pl
from jax.experimental.pallas import tpu as pltpu
from jax.experimental.pallas import tpu_sc as plsc

_NALLELES = 10
_D = 32
_L = 16
_NC = 2
_NS = 16
_NW = _NC * _NS

_GATHER_DNUMS = lax.GatherDimensionNumbers(
    offset_dims=(), collapsed_slice_dims=(0,), start_index_map=(0,))


def _lane_gather(src, idx):
  return lax.gather(src, idx[:, None], _GATHER_DNUMS, slice_sizes=(1,),
                    mode=lax.GatherScatterMode.PROMISE_IN_BOUNDS)


def _sc_embed(packed, table, n_rows, chunk):
  per_w = n_rows // _NW
  n_chunks = per_w // chunk
  assert per_w % chunk == 0 and n_chunks % 2 == 0 and n_chunks >= 8

  mesh = plsc.VectorSubcoreMesh(core_axis_name="c", subcore_axis_name="s")

  @functools.partial(
      pl.kernel,
      mesh=mesh,
      out_type=jax.ShapeDtypeStruct((n_rows // 200, 200, _D), jnp.float32),
      compiler_params=pltpu.CompilerParams(use_tc_tiling_on_sc=False),
      scratch_types=(
          [pltpu.VMEM((chunk,), jnp.int32)] * 2
          + [pltpu.VMEM((2 * chunk,), jnp.int32)] * 2
          + [pltpu.VMEM((2 * chunk, _D), jnp.float32)] * 2
          + [pltpu.VMEM((chunk // 200, 200, _D), jnp.float32)] * 2
          + [pltpu.SemaphoreType.DMA] * 6),
  )
  def k(w_hbm, table_hbm, out_hbm,
        w0, w1, ix0, ix1, g0, g1, o0, o1,
        isem0, isem1, gsem0, gsem1, wsem0, wsem1):
    w_v = (w0, w1)
    ix_v = (ix0, ix1)
    g_v = (g0, g1)
    o_v = (o0, o1)
    isem = (isem0, isem1)
    gsem = (gsem0, gsem1)
    wsem = (wsem0, wsem1)

    wid = lax.axis_index("s") * _NC + lax.axis_index("c")
    w_base = wid * per_w

    lane = lax.iota(jnp.int32, _L)
    dup_lo = lax.shift_right_logical(lane, 1)
    dup_hi = dup_lo + 8
    a_shift = (1 - (lane & 1)) * 4

    def issue_in(t, b):
      base = w_base + t * chunk
      pltpu.async_copy(w_hbm.at[pl.ds(base, chunk)], w_v[b], isem[b])

    def wait_in(b):
      pltpu.make_async_copy(w_hbm.at[pl.ds(0, chunk)], w_v[b],
                            isem[b]).wait()

    def compute_idx(b):
      def body(j, _):
        s = j * _L
        w = w_v[b][pl.ds(s, _L)]
        wlo = _lane_gather(w, dup_lo)
        whi = _lane_gather(w, dup_hi)
        ixlo = (lax.shift_right_logical(wlo, 8) * _NALLELES
                + (lax.shift_right_logical(wlo, a_shift) & 15))
        ixhi = (lax.shift_right_logical(whi, 8) * _NALLELES
                + (lax.shift_right_logical(whi, a_shift) & 15))
        ix_v[b][pl.ds(2 * s, _L)] = ixlo
        ix_v[b][pl.ds(2 * s + _L, _L)] = ixhi
        return 0

      lax.fori_loop(0, chunk // _L, body, 0, unroll=4)

    def issue_gather(b):
      pltpu.async_copy(table_hbm.at[ix_v[b]], g_v[b], gsem[b])

    def wait_gather(b):
      pltpu.make_async_copy(table_hbm.at[pl.ds(0, 2 * chunk)], g_v[b],
                            gsem[b]).wait()

    def add_rows(b):
      for bb in range(chunk // 200):
        def body(ss, _):
          i2 = 2 * (bb * 200 + ss)
          o_v[b][bb, ss, pl.ds(0, _L)] = (
              g_v[b][i2, pl.ds(0, _L)] + g_v[b][i2 + 1, pl.ds(0, _L)])
          o_v[b][bb, ss, pl.ds(_L, _L)] = (
              g_v[b][i2, pl.ds(_L, _L)] + g_v[b][i2 + 1, pl.ds(_L, _L)])
          return 0

        lax.fori_loop(0, 200, body, 0, unroll=4)

    def issue_wb(t, b):
      bbase = (w_base + t * chunk) // 200
      pltpu.async_copy(o_v[b], out_hbm.at[pl.ds(bbase, chunk // 200)],
                       wsem[b])

    def wait_wb(b):
      pltpu.make_async_copy(o_v[b], out_hbm.at[pl.ds(0, chunk // 200)],
                            wsem[b]).wait()

    issue_in(0, 0)
    issue_in(1, 1)
    wait_in(0)
    compute_idx(0)
    issue_in(2, 0)
    issue_gather(0)
    wait_in(1)
    compute_idx(1)
    issue_in(3, 1)
    issue_gather(1)
    wait_gather(0)
    add_rows(0)
    issue_wb(0, 0)
    wait_in(0)
    compute_idx(0)
    issue_in(4, 0)
    issue_gather(0)
    wait_gather(1)
    add_rows(1)
    issue_wb(1, 1)
    wait_in(1)
    compute_idx(1)
    issue_in(5, 1)
    issue_gather(1)
    wait_gather(0)
    wait_wb(0)
    add_rows(0)
    issue_wb(2, 0)

    def pair_body(g, _):
      for b in range(2):
        t = 2 * g + b
        wait_in(b)
        compute_idx(b)
        issue_in(t + 2, b)
        issue_gather(b)
        wait_gather(1 - b)
        wait_wb(1 - b)
        add_rows(1 - b)
        issue_wb(t - 1, 1 - b)
      return 0

    lax.fori_loop(2, n_chunks // 2 - 1, pair_body, 0)

    tl = n_chunks - 2
    wait_in(0)
    compute_idx(0)
    issue_gather(0)
    wait_gather(1)
    wait_wb(1)
    add_rows(1)
    issue_wb(tl - 1, 1)

    wait_in(1)
    compute_idx(1)
    issue_gather(1)
    wait_gather(0)
    wait_wb(0)
    add_rows(0)
    issue_wb(tl, 0)

    wait_gather(1)
    wait_wb(1)
    add_rows(1)
    issue_wb(tl + 1, 1)
    wait_wb(0)
    wait_wb(1)

  return k(packed, table)


def kernel(alleles, positions, table):
  b, s, _ = alleles.shape
  n = b * s
  packed = (
      lax.shift_left(positions.astype(jnp.int32), 8)
      | lax.shift_left(alleles[:, :, 0].astype(jnp.int32), 4)
      | alleles[:, :, 1].astype(jnp.int32)
  ).reshape(n)
  out = _sc_embed(packed, table, n, 400)
  return out


# speedup vs baseline: 2.1013x; 1.1678x over previous
import functools

import jax
import jax.numpy as jnp
from jax import lax
from jax.experimental import pallas as pl
from jax.experimental.pallas import tpu as pltpu
from jax.experimental.pallas import tpu_sc as plsc

_NALLELES = 10
_D = 32
_L = 16
_NC = 2
_NS = 16
_NW = _NC * _NS
_NB = 3


def _sc_embed(packed, table, n_rows, chunk):
  per_w = n_rows // _NW
  n_chunks = per_w // chunk
  assert per_w % chunk == 0
  n_triples = (n_chunks - 8) // _NB
  assert n_triples * _NB == n_chunks - 8 and n_triples >= 1

  mesh = plsc.VectorSubcoreMesh(core_axis_name="c", subcore_axis_name="s")

  @functools.partial(
      pl.kernel,
      mesh=mesh,
      out_type=jax.ShapeDtypeStruct((n_rows, _D), jnp.float32),
      compiler_params=pltpu.CompilerParams(use_tc_tiling_on_sc=False),
      scratch_types=(
          [pltpu.VMEM((chunk,), jnp.int32)] * _NB
          + [pltpu.VMEM((chunk,), jnp.int32)] * _NB
          + [pltpu.VMEM((chunk,), jnp.int32)] * _NB
          + [pltpu.VMEM((chunk, _D), jnp.float32)] * _NB
          + [pltpu.SemaphoreType.DMA] * (4 * _NB)),
  )
  def k(w_hbm, table_hbm, out_hbm,
        w0, w1, w2, x0, x1, x2, y0, y1, y2, o0, o1, o2,
        is0, is1, is2, g0s0, g0s1, g0s2, g1s0, g1s1, g1s2,
        ws0, ws1, ws2):
    w_v = (w0, w1, w2)
    ix0_v = (x0, x1, x2)
    ix1_v = (y0, y1, y2)
    o_v = (o0, o1, o2)
    isem = (is0, is1, is2)
    g0sem = (g0s0, g0s1, g0s2)
    g1sem = (g1s0, g1s1, g1s2)
    wsem = (ws0, ws1, ws2)

    wid = lax.axis_index("s") * _NC + lax.axis_index("c")
    w_base = wid * per_w

    def issue_in(t, r):
      base = w_base + t * chunk
      pltpu.async_copy(w_hbm.at[pl.ds(base, chunk)], w_v[r], isem[r])

    def wait_in(r):
      pltpu.make_async_copy(w_hbm.at[pl.ds(0, chunk)], w_v[r],
                            isem[r]).wait()

    def compute_idx(r):
      def body(j, _):
        s = j * _L
        w = w_v[r][pl.ds(s, _L)]
        p = lax.shift_right_logical(w, 8) * _NALLELES
        ix0_v[r][pl.ds(s, _L)] = (
            p + (lax.shift_right_logical(w, 4) & 15))
        ix1_v[r][pl.ds(s, _L)] = p + (w & 15)
        return 0

      lax.fori_loop(0, chunk // _L, body, 0, unroll=4)

    def issue_g0(r):
      pltpu.async_copy(table_hbm.at[ix0_v[r]], o_v[r], g0sem[r])

    def wait_g0(r):
      pltpu.make_async_copy(table_hbm.at[pl.ds(0, chunk)], o_v[r],
                            g0sem[r]).wait()

    def issue_g1(r):
      pltpu.async_copy(table_hbm.at[ix1_v[r]], o_v[r], g1sem[r], add=True)

    def wait_g1(r):
      pltpu.make_async_copy(table_hbm.at[pl.ds(0, chunk)], o_v[r],
                            g1sem[r]).wait()

    def issue_wb(t, r):
      base = w_base + t * chunk
      pltpu.async_copy(o_v[r], out_hbm.at[pl.ds(base, chunk)], wsem[r])

    def wait_wb(r):
      pltpu.make_async_copy(o_v[r], out_hbm.at[pl.ds(0, chunk)],
                            wsem[r]).wait()

    def step(t, r, do_in=True, do_g1=True, do_wb=True, do_wbwait=True):
      r1 = (r + 2) % _NB
      r2 = (r + 1) % _NB
      wait_in(r)
      compute_idx(r)
      if do_in:
        issue_in(t + _NB, r)
      if do_g1:
        wait_g0(r1)
        issue_g1(r1)
      if do_wb:
        wait_g1(r2)
        issue_wb(t - 2, r2)
      if do_wbwait:
        wait_wb(r)
      issue_g0(r)

    issue_in(0, 0)
    issue_in(1, 1)
    issue_in(2, 2)
    step(0, 0, do_g1=False, do_wb=False, do_wbwait=False)
    step(1, 1, do_wb=False, do_wbwait=False)
    step(2, 2, do_wbwait=False)

    def triple(g, _):
      for r in range(_NB):
        step(_NB * g + r, r)
      return 0

    lax.fori_loop(1, n_triples + 1, triple, 0)

    tt = n_chunks - 5
    step(tt, tt % _NB)
    step(tt + 1, (tt + 1) % _NB)
    step(tt + 2, (tt + 2) % _NB, do_in=False)
    step(tt + 3, (tt + 3) % _NB, do_in=False)
    step(tt + 4, (tt + 4) % _NB, do_in=False)

    tl = n_chunks - 1
    rl = tl % _NB
    rp = (tl - 1) % _NB
    wait_g0(rl)
    issue_g1(rl)
    wait_g1(rp)
    issue_wb(tl - 1, rp)
    wait_g1(rl)
    issue_wb(tl, rl)
    wait_wb((tl + 1) % _NB)
    wait_wb(rp)
    wait_wb(rl)

  return k(packed, table)


def kernel(alleles, positions, table):
  b, s, _ = alleles.shape
  n = b * s
  packed = (
      lax.shift_left(positions.astype(jnp.int32), 8)
      | lax.shift_left(alleles[:, :, 0].astype(jnp.int32), 4)
      | alleles[:, :, 1].astype(jnp.int32)
  ).reshape(n)
  out = _sc_embed(packed, table, n, 800)
  return out.reshape(b, s, _D)
